# initial kernel scaffold (unmeasured)
import jax
import jax.numpy as jnp
from jax import lax
from jax.experimental import pallas as pl
from jax.experimental.pallas import tpu as pltpu


def kernel(
    x,
):
    def body(*refs):
        pass

    out_shape = jax.ShapeDtypeStruct(..., jnp.float32)
    return pl.pallas_call(body, out_shape=out_shape)(...)


# baseline (device time: 2706407 ns/iter reference)
import jax
import jax.numpy as jnp
from jax import lax
from jax.experimental import pallas as pl
from jax.experimental.pallas import tpu as pltpu

NZ = 4
M, N = 16384, 1024
CH = M // NZ
NSLOT = 4


def kernel(x):
    def body(x_ref, out_ref, comm_ref, acc_ref, xc_ref,
             send_sems, recv_sems, copy_sems):
        my_x = lax.axis_index("x")
        my_y = lax.axis_index("y")
        my_z = lax.axis_index("z")
        right = (my_z + 1) % NZ
        left = (my_z + NZ - 1) % NZ

        barrier = pltpu.get_barrier_semaphore()
        for nbr in (left, right):
            pl.semaphore_signal(
                barrier, inc=1,
                device_id=(my_x, my_y, nbr),
                device_id_type=pl.DeviceIdType.MESH,
            )
        pl.semaphore_wait(barrier, 2)

        cp = pltpu.make_async_copy(
            x_ref.at[pl.ds(my_z * CH, CH), :], acc_ref, copy_sems.at[0])
        cp.start()
        cp.wait()

        for h in range(NZ - 1):
            slot = h % NSLOT
            rc = (my_z - h - 1) % NZ
            rdma = pltpu.make_async_remote_copy(
                src_ref=acc_ref,
                dst_ref=comm_ref.at[slot],
                send_sem=send_sems.at[slot],
                recv_sem=recv_sems.at[slot],
                device_id=(my_x, my_y, right),
                device_id_type=pl.DeviceIdType.MESH,
            )
            rdma.start()
            cpx = pltpu.make_async_copy(
                x_ref.at[pl.ds(rc * CH, CH), :], xc_ref, copy_sems.at[1])
            cpx.start()
            rdma.wait()
            cpx.wait()
            cpr = pltpu.make_async_copy(
                comm_ref.at[slot], acc_ref, copy_sems.at[0])
            cpr.start()
            cpr.wait()
            acc_ref[...] = acc_ref[...] + xc_ref[...]

        r0 = (my_z + 1) % NZ
        cpo = pltpu.make_async_copy(
            acc_ref, out_ref.at[pl.ds(r0 * CH, CH), :], copy_sems.at[0])
        cpo.start()
        cpo.wait()

        for k in range(NZ - 1):
            h = k + NZ - 1
            slot = h % NSLOT
            src = acc_ref if k == 0 else comm_ref.at[(h - 1) % NSLOT]
            rdma = pltpu.make_async_remote_copy(
                src_ref=src,
                dst_ref=comm_ref.at[slot],
                send_sem=send_sems.at[slot],
                recv_sem=recv_sems.at[slot],
                device_id=(my_x, my_y, right),
                device_id_type=pl.DeviceIdType.MESH,
            )
            rdma.start()
            rdma.wait()
            rcg = (my_z - k) % NZ
            cpo = pltpu.make_async_copy(
                comm_ref.at[slot],
                out_ref.at[pl.ds(rcg * CH, CH), :],
                copy_sems.at[0])
            cpo.start()
            cpo.wait()

    out, _comm = pl.pallas_call(
        body,
        out_shape=[
            jax.ShapeDtypeStruct((M, N), jnp.float32),
            jax.ShapeDtypeStruct((NSLOT, CH, N), jnp.float32),
        ],
        in_specs=[pl.BlockSpec(memory_space=pl.ANY)],
        out_specs=[
            pl.BlockSpec(memory_space=pl.ANY),
            pl.BlockSpec(memory_space=pl.ANY),
        ],
        scratch_shapes=[
            pltpu.VMEM((CH, N), jnp.float32),
            pltpu.VMEM((CH, N), jnp.float32),
            pltpu.SemaphoreType.DMA((NSLOT,)),
            pltpu.SemaphoreType.DMA((NSLOT,)),
            pltpu.SemaphoreType.DMA((2,)),
        ],
        compiler_params=pltpu.CompilerParams(collective_id=0),
    )(x)
    return out


# device time: 1174431 ns/iter; 2.3044x vs baseline; 2.3044x over previous
import jax
import jax.numpy as jnp
from jax import lax
from jax.experimental import pallas as pl
from jax.experimental.pallas import tpu as pltpu

NZ = 4
M, N = 16384, 1024
CH = M // NZ
NSLOT = 4


def kernel(x):
    def body(x_ref, out_ref, comm_ref, acc_ref, xc_ref,
             send_sems, recv_sems, copy_sems):
        my_x = lax.axis_index("x")
        my_y = lax.axis_index("y")
        my_z = lax.axis_index("z")
        right = (my_z + 1) % NZ
        left = (my_z + NZ - 1) % NZ

        barrier = pltpu.get_barrier_semaphore()
        for nbr in (left, right):
            pl.semaphore_signal(
                barrier, inc=1,
                device_id=(my_x, my_y, nbr),
                device_id_type=pl.DeviceIdType.MESH,
            )
        pl.semaphore_wait(barrier, 2)

        cp = pltpu.make_async_copy(
            x_ref.at[pl.ds(my_z * CH, CH), :], acc_ref, copy_sems.at[0])
        cp.start()
        cp.wait()

        for h in range(NZ - 1):
            slot = h % NSLOT
            rc = (my_z - h - 1) % NZ
            rdma = pltpu.make_async_remote_copy(
                src_ref=acc_ref,
                dst_ref=comm_ref.at[slot],
                send_sem=send_sems.at[slot],
                recv_sem=recv_sems.at[slot],
                device_id=(my_x, my_y, right),
                device_id_type=pl.DeviceIdType.MESH,
            )
            rdma.start()
            cpx = pltpu.make_async_copy(
                x_ref.at[pl.ds(rc * CH, CH), :], xc_ref, copy_sems.at[1])
            cpx.start()
            rdma.wait()
            cpx.wait()
            cpr = pltpu.make_async_copy(
                comm_ref.at[slot], acc_ref, copy_sems.at[0])
            cpr.start()
            cpr.wait()
            acc_ref[...] = acc_ref[...] + xc_ref[...]

        r0 = (my_z + 1) % NZ
        cpo = pltpu.make_async_copy(
            acc_ref, out_ref.at[pl.ds(r0 * CH, CH), :], copy_sems.at[0])
        cpo.start()
        cpo.wait()

        for k in range(NZ - 1):
            h = k + NZ - 1
            slot = h % NSLOT
            sc = (my_z + 1 - k) % NZ
            src = (acc_ref if k == 0
                   else out_ref.at[pl.ds(sc * CH, CH), :])
            rdma = pltpu.make_async_remote_copy(
                src_ref=src,
                dst_ref=out_ref.at[pl.ds(sc * CH, CH), :],
                send_sem=send_sems.at[slot],
                recv_sem=recv_sems.at[slot],
                device_id=(my_x, my_y, right),
                device_id_type=pl.DeviceIdType.MESH,
            )
            rdma.start()
            rdma.wait()

    out, _comm = pl.pallas_call(
        body,
        out_shape=[
            jax.ShapeDtypeStruct((M, N), jnp.float32),
            jax.ShapeDtypeStruct((NSLOT, CH, N), jnp.float32),
        ],
        in_specs=[pl.BlockSpec(memory_space=pl.ANY)],
        out_specs=[
            pl.BlockSpec(memory_space=pl.ANY),
            pl.BlockSpec(memory_space=pl.ANY),
        ],
        scratch_shapes=[
            pltpu.VMEM((CH, N), jnp.float32),
            pltpu.VMEM((CH, N), jnp.float32),
            pltpu.SemaphoreType.DMA((NSLOT,)),
            pltpu.SemaphoreType.DMA((NSLOT,)),
            pltpu.SemaphoreType.DMA((2,)),
        ],
        compiler_params=pltpu.CompilerParams(collective_id=0),
    )(x)
    return out


# device time: 611181 ns/iter; 4.4282x vs baseline; 1.9216x over previous
import jax
import jax.numpy as jnp
from jax import lax
from jax.experimental import pallas as pl
from jax.experimental.pallas import tpu as pltpu

NZ = 4
M, N = 16384, 1024
QR = M // 4
CR = QR // NZ
NSLOT = 4


def kernel(x):
    def body(x_ref, out_ref, qcomm_ref, qacc_ref, qxc_ref,
             z_send, z_recv, xy_send, xy_recv, copy_sems):
        my_x = lax.axis_index("x")
        my_y = lax.axis_index("y")
        my_z = lax.axis_index("z")
        zright = (my_z + 1) % NZ
        zleft = (my_z + NZ - 1) % NZ

        q = my_x * 2 + my_y
        qx = (1 - my_x) * 2 + my_y
        qy = my_x * 2 + (1 - my_y)
        qd = (1 - my_x) * 2 + (1 - my_y)
        q0 = q * QR

        barrier = pltpu.get_barrier_semaphore()
        for dev in ((my_x, my_y, zleft), (my_x, my_y, zright),
                    (1 - my_x, my_y, my_z), (my_x, 1 - my_y, my_z)):
            pl.semaphore_signal(barrier, inc=1, device_id=dev,
                                device_id_type=pl.DeviceIdType.MESH)
        pl.semaphore_wait(barrier, 4)

        cp = pltpu.make_async_copy(
            x_ref.at[pl.ds(q0 + my_z * CR, CR), :], qacc_ref,
            copy_sems.at[0])
        cp.start()
        cp.wait()

        for h in range(NZ - 1):
            slot = h % NSLOT
            rc = (my_z - h - 1) % NZ
            rdma = pltpu.make_async_remote_copy(
                src_ref=qacc_ref,
                dst_ref=qcomm_ref.at[slot],
                send_sem=z_send.at[slot],
                recv_sem=z_recv.at[slot],
                device_id=(my_x, my_y, zright),
                device_id_type=pl.DeviceIdType.MESH,
            )
            rdma.start()
            cpx = pltpu.make_async_copy(
                x_ref.at[pl.ds(q0 + rc * CR, CR), :], qxc_ref,
                copy_sems.at[1])
            cpx.start()
            rdma.wait()
            cpx.wait()
            qacc_ref[...] = qcomm_ref[slot] + qxc_ref[...]

        rz = (my_z + 1) % NZ
        cpo = pltpu.make_async_copy(
            qacc_ref, out_ref.at[pl.ds(q0 + rz * CR, CR), :],
            copy_sems.at[0])
        cpo.start()
        cpo.wait()

        for k in range(NZ - 1):
            h = k + NZ - 1
            slot = h % NSLOT
            sc = (my_z + 1 - k) % NZ
            row = q0 + sc * CR
            src = (qacc_ref if k == 0
                   else out_ref.at[pl.ds(row, CR), :])
            rdma = pltpu.make_async_remote_copy(
                src_ref=src,
                dst_ref=out_ref.at[pl.ds(row, CR), :],
                send_sem=z_send.at[slot],
                recv_sem=z_recv.at[slot],
                device_id=(my_x, my_y, zright),
                device_id_type=pl.DeviceIdType.MESH,
            )
            rdma.start()
            rdma.wait()

        rdx = pltpu.make_async_remote_copy(
            src_ref=out_ref.at[pl.ds(q0, QR), :],
            dst_ref=out_ref.at[pl.ds(q0, QR), :],
            send_sem=xy_send.at[0],
            recv_sem=xy_recv.at[0],
            device_id=(1 - my_x, my_y, my_z),
            device_id_type=pl.DeviceIdType.MESH,
        )
        rdy = pltpu.make_async_remote_copy(
            src_ref=out_ref.at[pl.ds(q0, QR), :],
            dst_ref=out_ref.at[pl.ds(q0, QR), :],
            send_sem=xy_send.at[1],
            recv_sem=xy_recv.at[1],
            device_id=(my_x, 1 - my_y, my_z),
            device_id_type=pl.DeviceIdType.MESH,
        )
        rdx.start()
        rdy.start()
        rdx.wait()
        rdy.wait()

        H = QR // 2
        rdx2 = pltpu.make_async_remote_copy(
            src_ref=out_ref.at[pl.ds(qy * QR, H), :],
            dst_ref=out_ref.at[pl.ds(qy * QR, H), :],
            send_sem=xy_send.at[2],
            recv_sem=xy_recv.at[2],
            device_id=(1 - my_x, my_y, my_z),
            device_id_type=pl.DeviceIdType.MESH,
        )
        rdy2 = pltpu.make_async_remote_copy(
            src_ref=out_ref.at[pl.ds(qx * QR + H, H), :],
            dst_ref=out_ref.at[pl.ds(qx * QR + H, H), :],
            send_sem=xy_send.at[3],
            recv_sem=xy_recv.at[3],
            device_id=(my_x, 1 - my_y, my_z),
            device_id_type=pl.DeviceIdType.MESH,
        )
        rdx2.start()
        rdy2.start()
        rdx2.wait()
        rdy2.wait()

    return pl.pallas_call(
        body,
        out_shape=jax.ShapeDtypeStruct((M, N), jnp.float32),
        in_specs=[pl.BlockSpec(memory_space=pl.ANY)],
        out_specs=pl.BlockSpec(memory_space=pl.ANY),
        scratch_shapes=[
            pltpu.VMEM((NSLOT, CR, N), jnp.float32),
            pltpu.VMEM((CR, N), jnp.float32),
            pltpu.VMEM((CR, N), jnp.float32),
            pltpu.SemaphoreType.DMA((NSLOT,)),
            pltpu.SemaphoreType.DMA((NSLOT,)),
            pltpu.SemaphoreType.DMA((4,)),
            pltpu.SemaphoreType.DMA((4,)),
            pltpu.SemaphoreType.DMA((2,)),
        ],
        compiler_params=pltpu.CompilerParams(collective_id=0),
    )(x)


# device time: 476706 ns/iter; 5.6773x vs baseline; 1.2821x over previous
import jax
import jax.numpy as jnp
from jax import lax
from jax.experimental import pallas as pl
from jax.experimental.pallas import tpu as pltpu

NZ = 4
M, N = 16384, 1024
QR = M // 4
CR = QR // NZ
NSLOT = 4


def kernel(x):
    def body(x_ref, out_ref, qcomm_ref, qacc_ref, qxc_ref,
             z_send, z_recv, xy_send, xy_recv, p2_send, p2_recv,
             copy_sems):
        my_x = lax.axis_index("x")
        my_y = lax.axis_index("y")
        my_z = lax.axis_index("z")
        zright = (my_z + 1) % NZ
        zleft = (my_z + NZ - 1) % NZ

        q = my_x * 2 + my_y
        qx = (1 - my_x) * 2 + my_y
        qy = my_x * 2 + (1 - my_y)
        qd = (1 - my_x) * 2 + (1 - my_y)
        q0 = q * QR

        barrier = pltpu.get_barrier_semaphore()
        for dev in ((my_x, my_y, zleft), (my_x, my_y, zright),
                    (1 - my_x, my_y, my_z), (my_x, 1 - my_y, my_z)):
            pl.semaphore_signal(barrier, inc=1, device_id=dev,
                                device_id_type=pl.DeviceIdType.MESH)
        pl.semaphore_wait(barrier, 4)

        cp = pltpu.make_async_copy(
            x_ref.at[pl.ds(q0 + my_z * CR, CR), :], qacc_ref,
            copy_sems.at[0])
        cp.start()
        cp.wait()

        for h in range(NZ - 1):
            slot = h % NSLOT
            rc = (my_z - h - 1) % NZ
            rdma = pltpu.make_async_remote_copy(
                src_ref=qacc_ref,
                dst_ref=qcomm_ref.at[slot],
                send_sem=z_send.at[slot],
                recv_sem=z_recv.at[slot],
                device_id=(my_x, my_y, zright),
                device_id_type=pl.DeviceIdType.MESH,
            )
            rdma.start()
            cpx = pltpu.make_async_copy(
                x_ref.at[pl.ds(q0 + rc * CR, CR), :], qxc_ref,
                copy_sems.at[1])
            cpx.start()
            rdma.wait()
            cpx.wait()
            qacc_ref[...] = qcomm_ref[slot] + qxc_ref[...]

        rz = (my_z + 1) % NZ
        cpo = pltpu.make_async_copy(
            qacc_ref, out_ref.at[pl.ds(q0 + rz * CR, CR), :],
            copy_sems.at[0])
        cpo.start()
        cpo.wait()

        xy_pending = []
        for r in range(NZ):
            cr = (my_z + 1 - r) % NZ
            row = q0 + cr * CR
            ag = None
            if r < NZ - 1:
                slot = (r + NZ - 1) % NSLOT
                src = (qacc_ref if r == 0
                       else out_ref.at[pl.ds(row, CR), :])
                ag = pltpu.make_async_remote_copy(
                    src_ref=src,
                    dst_ref=out_ref.at[pl.ds(row, CR), :],
                    send_sem=z_send.at[slot],
                    recv_sem=z_recv.at[slot],
                    device_id=(my_x, my_y, zright),
                    device_id_type=pl.DeviceIdType.MESH,
                )
                ag.start()
            for li, dev in enumerate(((1 - my_x, my_y, my_z),
                                      (my_x, 1 - my_y, my_z))):
                si = 2 * r + li
                xy = pltpu.make_async_remote_copy(
                    src_ref=out_ref.at[pl.ds(row, CR), :],
                    dst_ref=out_ref.at[pl.ds(row, CR), :],
                    send_sem=xy_send.at[si],
                    recv_sem=xy_recv.at[si],
                    device_id=dev,
                    device_id_type=pl.DeviceIdType.MESH,
                )
                xy.start()
                xy_pending.append(xy)
            if ag is not None:
                ag.wait()

        for xy in xy_pending:
            xy.wait()

        H = QR // 2
        rdx2 = pltpu.make_async_remote_copy(
            src_ref=out_ref.at[pl.ds(qy * QR, H), :],
            dst_ref=out_ref.at[pl.ds(qy * QR, H), :],
            send_sem=p2_send.at[0],
            recv_sem=p2_recv.at[0],
            device_id=(1 - my_x, my_y, my_z),
            device_id_type=pl.DeviceIdType.MESH,
        )
        rdy2 = pltpu.make_async_remote_copy(
            src_ref=out_ref.at[pl.ds(qx * QR + H, H), :],
            dst_ref=out_ref.at[pl.ds(qx * QR + H, H), :],
            send_sem=p2_send.at[1],
            recv_sem=p2_recv.at[1],
            device_id=(my_x, 1 - my_y, my_z),
            device_id_type=pl.DeviceIdType.MESH,
        )
        rdx2.start()
        rdy2.start()
        rdx2.wait()
        rdy2.wait()

    return pl.pallas_call(
        body,
        out_shape=jax.ShapeDtypeStruct((M, N), jnp.float32),
        in_specs=[pl.BlockSpec(memory_space=pl.ANY)],
        out_specs=pl.BlockSpec(memory_space=pl.ANY),
        scratch_shapes=[
            pltpu.VMEM((NSLOT, CR, N), jnp.float32),
            pltpu.VMEM((CR, N), jnp.float32),
            pltpu.VMEM((CR, N), jnp.float32),
            pltpu.SemaphoreType.DMA((NSLOT,)),
            pltpu.SemaphoreType.DMA((NSLOT,)),
            pltpu.SemaphoreType.DMA((8,)),
            pltpu.SemaphoreType.DMA((8,)),
            pltpu.SemaphoreType.DMA((2,)),
            pltpu.SemaphoreType.DMA((2,)),
            pltpu.SemaphoreType.DMA((2,)),
        ],
        compiler_params=pltpu.CompilerParams(collective_id=0),
    )(x)


# device time: 469913 ns/iter; 5.7594x vs baseline; 1.0145x over previous
import jax
import jax.numpy as jnp
from jax import lax
from jax.experimental import pallas as pl
from jax.experimental.pallas import tpu as pltpu

NZ = 4
M, N = 16384, 1024
QR = M // 4
CR = QR // NZ
NSLOT = 4


def kernel(x):
    def body(x_ref, out_ref, qcomm_ref, qacc_ref, qxc_ref,
             z_send, z_recv, xy_send, xy_recv, p2_send, p2_recv,
             copy_sems):
        my_x = lax.axis_index("x")
        my_y = lax.axis_index("y")
        my_z = lax.axis_index("z")
        zright = (my_z + 1) % NZ
        zleft = (my_z + NZ - 1) % NZ

        q = my_x * 2 + my_y
        qx = (1 - my_x) * 2 + my_y
        qy = my_x * 2 + (1 - my_y)
        qd = (1 - my_x) * 2 + (1 - my_y)
        q0 = q * QR

        barrier = pltpu.get_barrier_semaphore()
        for dev in ((my_x, my_y, zleft), (my_x, my_y, zright),
                    (1 - my_x, my_y, my_z), (my_x, 1 - my_y, my_z)):
            pl.semaphore_signal(barrier, inc=1, device_id=dev,
                                device_id_type=pl.DeviceIdType.MESH)
        pl.semaphore_wait(barrier, 4)

        cp = pltpu.make_async_copy(
            x_ref.at[pl.ds(q0 + my_z * CR, CR), :], qacc_ref,
            copy_sems.at[0])
        cp.start()
        cp.wait()

        for h in range(NZ - 1):
            slot = h % NSLOT
            rc = (my_z - h - 1) % NZ
            rdma = pltpu.make_async_remote_copy(
                src_ref=qacc_ref,
                dst_ref=qcomm_ref.at[slot],
                send_sem=z_send.at[slot],
                recv_sem=z_recv.at[slot],
                device_id=(my_x, my_y, zright),
                device_id_type=pl.DeviceIdType.MESH,
            )
            rdma.start()
            cpx = pltpu.make_async_copy(
                x_ref.at[pl.ds(q0 + rc * CR, CR), :], qxc_ref,
                copy_sems.at[1])
            cpx.start()
            rdma.wait()
            cpx.wait()
            qacc_ref[...] = qcomm_ref[slot] + qxc_ref[...]

        rz = (my_z + 1) % NZ
        cpo = pltpu.make_async_copy(
            qacc_ref, out_ref.at[pl.ds(q0 + rz * CR, CR), :],
            copy_sems.at[0])
        cpo.start()
        cpo.wait()

        Hh = CR // 2
        p1 = {}
        relays = []
        xdev = (1 - my_x, my_y, my_z)
        ydev = (my_x, 1 - my_y, my_z)

        def relay(j):
            cj = (my_z + 1 - j) % NZ
            p1[(j, 0)].wait_recv()
            p1[(j, 1)].wait_recv()
            xrow = qy * QR + cj * CR
            yrow = qx * QR + cj * CR + Hh
            for li, (dev, row) in enumerate(((xdev, xrow), (ydev, yrow))):
                si = 2 * j + li
                rl = pltpu.make_async_remote_copy(
                    src_ref=out_ref.at[pl.ds(row, Hh), :],
                    dst_ref=out_ref.at[pl.ds(row, Hh), :],
                    send_sem=p2_send.at[si],
                    recv_sem=p2_recv.at[si],
                    device_id=dev,
                    device_id_type=pl.DeviceIdType.MESH,
                )
                rl.start()
                relays.append(rl)

        for r in range(NZ):
            cr = (my_z + 1 - r) % NZ
            row = q0 + cr * CR
            ag = None
            if r < NZ - 1:
                slot = (r + NZ - 1) % NSLOT
                src = (qacc_ref if r == 0
                       else out_ref.at[pl.ds(row, CR), :])
                ag = pltpu.make_async_remote_copy(
                    src_ref=src,
                    dst_ref=out_ref.at[pl.ds(row, CR), :],
                    send_sem=z_send.at[slot],
                    recv_sem=z_recv.at[slot],
                    device_id=(my_x, my_y, zright),
                    device_id_type=pl.DeviceIdType.MESH,
                )
                ag.start()
            for li, dev in enumerate((xdev, ydev)):
                si = 2 * r + li
                xy = pltpu.make_async_remote_copy(
                    src_ref=out_ref.at[pl.ds(row, CR), :],
                    dst_ref=out_ref.at[pl.ds(row, CR), :],
                    send_sem=xy_send.at[si],
                    recv_sem=xy_recv.at[si],
                    device_id=dev,
                    device_id_type=pl.DeviceIdType.MESH,
                )
                xy.start()
                p1[(r, li)] = xy
            if r > 0:
                relay(r - 1)
            if ag is not None:
                ag.wait()

        relay(NZ - 1)

        for xy in p1.values():
            xy.wait_send()
        for rl in relays:
            rl.wait()

    return pl.pallas_call(
        body,
        out_shape=jax.ShapeDtypeStruct((M, N), jnp.float32),
        in_specs=[pl.BlockSpec(memory_space=pl.ANY)],
        out_specs=pl.BlockSpec(memory_space=pl.ANY),
        scratch_shapes=[
            pltpu.VMEM((NSLOT, CR, N), jnp.float32),
            pltpu.VMEM((CR, N), jnp.float32),
            pltpu.VMEM((CR, N), jnp.float32),
            pltpu.SemaphoreType.DMA((NSLOT,)),
            pltpu.SemaphoreType.DMA((NSLOT,)),
            pltpu.SemaphoreType.DMA((8,)),
            pltpu.SemaphoreType.DMA((8,)),
            pltpu.SemaphoreType.DMA((8,)),
            pltpu.SemaphoreType.DMA((8,)),
            pltpu.SemaphoreType.DMA((2,)),
        ],
        compiler_params=pltpu.CompilerParams(collective_id=0),
    )(x)


# device time: 464759 ns/iter; 5.8232x vs baseline; 1.0111x over previous
import jax
import jax.numpy as jnp
from jax import lax
from jax.experimental import pallas as pl
from jax.experimental.pallas import tpu as pltpu

NZ = 4
M, N = 16384, 1024
QR = M // 4
CR = QR // NZ
NSUB = 4
SC = CR // NSUB
NSLOT = 4


def kernel(x):
    def body(x_ref, out_ref, qcomm_ref, qacc_ref, qxc_ref,
             z_send, z_recv, ag_send, ag_recv, xy_send, xy_recv,
             p2_send, p2_recv, copy_sems):
        my_x = lax.axis_index("x")
        my_y = lax.axis_index("y")
        my_z = lax.axis_index("z")
        zright = (my_z + 1) % NZ
        zleft = (my_z + NZ - 1) % NZ

        q = my_x * 2 + my_y
        qx = (1 - my_x) * 2 + my_y
        qy = my_x * 2 + (1 - my_y)
        qd = (1 - my_x) * 2 + (1 - my_y)
        q0 = q * QR

        barrier = pltpu.get_barrier_semaphore()
        for dev in ((my_x, my_y, zleft), (my_x, my_y, zright),
                    (1 - my_x, my_y, my_z), (my_x, 1 - my_y, my_z)):
            pl.semaphore_signal(barrier, inc=1, device_id=dev,
                                device_id_type=pl.DeviceIdType.MESH)
        pl.semaphore_wait(barrier, 4)

        cp = pltpu.make_async_copy(
            x_ref.at[pl.ds(q0 + my_z * CR, CR), :], qacc_ref,
            copy_sems.at[0])
        cp.start()
        cp.wait()

        def rs_desc(h, s):
            return pltpu.make_async_remote_copy(
                src_ref=qacc_ref.at[pl.ds(s * SC, SC), :],
                dst_ref=qcomm_ref.at[h, s],
                send_sem=z_send.at[h * NSUB + s],
                recv_sem=z_recv.at[h * NSUB + s],
                device_id=(my_x, my_y, zright),
                device_id_type=pl.DeviceIdType.MESH,
            )

        desc = {}
        for s in range(NSUB):
            desc[(0, s)] = rs_desc(0, s)
            desc[(0, s)].start()
        cpx = pltpu.make_async_copy(
            x_ref.at[pl.ds(q0 + ((my_z - 1) % NZ) * CR, CR), :], qxc_ref,
            copy_sems.at[1])
        cpx.start()
        for h in range(NZ - 1):
            cpx.wait()
            for s in range(NSUB):
                desc[(h, s)].wait()
                qacc_ref[pl.ds(s * SC, SC), :] = (
                    qcomm_ref[h, s] + qxc_ref[pl.ds(s * SC, SC), :])
                if h < NZ - 2:
                    desc[(h + 1, s)] = rs_desc(h + 1, s)
                    desc[(h + 1, s)].start()
            if h < NZ - 2:
                rc = (my_z - h - 2) % NZ
                cpx = pltpu.make_async_copy(
                    x_ref.at[pl.ds(q0 + rc * CR, CR), :], qxc_ref,
                    copy_sems.at[h % 2])
                cpx.start()

        rz = (my_z + 1) % NZ
        cpo = pltpu.make_async_copy(
            qacc_ref, out_ref.at[pl.ds(q0 + rz * CR, CR), :],
            copy_sems.at[0])
        cpo.start()
        cpo.wait()

        Hh = CR // 2
        p1 = {}
        relays = []
        xdev = (1 - my_x, my_y, my_z)
        ydev = (my_x, 1 - my_y, my_z)

        def relay(j):
            cj = (my_z + 1 - j) % NZ
            p1[(j, 0)].wait_recv()
            p1[(j, 1)].wait_recv()
            xrow = qy * QR + cj * CR
            yrow = qx * QR + cj * CR + Hh
            for li, (dev, row) in enumerate(((xdev, xrow), (ydev, yrow))):
                si = 2 * j + li
                rl = pltpu.make_async_remote_copy(
                    src_ref=out_ref.at[pl.ds(row, Hh), :],
                    dst_ref=out_ref.at[pl.ds(row, Hh), :],
                    send_sem=p2_send.at[si],
                    recv_sem=p2_recv.at[si],
                    device_id=dev,
                    device_id_type=pl.DeviceIdType.MESH,
                )
                rl.start()
                relays.append(rl)

        for r in range(NZ):
            cr = (my_z + 1 - r) % NZ
            row = q0 + cr * CR
            ag = None
            if r < NZ - 1:
                src = (qacc_ref if r == 0
                       else out_ref.at[pl.ds(row, CR), :])
                ag = pltpu.make_async_remote_copy(
                    src_ref=src,
                    dst_ref=out_ref.at[pl.ds(row, CR), :],
                    send_sem=ag_send.at[r],
                    recv_sem=ag_recv.at[r],
                    device_id=(my_x, my_y, zright),
                    device_id_type=pl.DeviceIdType.MESH,
                )
                ag.start()
            for li, dev in enumerate((xdev, ydev)):
                si = 2 * r + li
                xy = pltpu.make_async_remote_copy(
                    src_ref=out_ref.at[pl.ds(row, CR), :],
                    dst_ref=out_ref.at[pl.ds(row, CR), :],
                    send_sem=xy_send.at[si],
                    recv_sem=xy_recv.at[si],
                    device_id=dev,
                    device_id_type=pl.DeviceIdType.MESH,
                )
                xy.start()
                p1[(r, li)] = xy
            if r > 0:
                relay(r - 1)
            if ag is not None:
                ag.wait()

        relay(NZ - 1)

        for xy in p1.values():
            xy.wait_send()
        for rl in relays:
            rl.wait()

    return pl.pallas_call(
        body,
        out_shape=jax.ShapeDtypeStruct((M, N), jnp.float32),
        in_specs=[pl.BlockSpec(memory_space=pl.ANY)],
        out_specs=pl.BlockSpec(memory_space=pl.ANY),
        scratch_shapes=[
            pltpu.VMEM((NZ - 1, NSUB, SC, N), jnp.float32),
            pltpu.VMEM((CR, N), jnp.float32),
            pltpu.VMEM((CR, N), jnp.float32),
            pltpu.SemaphoreType.DMA(((NZ - 1) * NSUB,)),
            pltpu.SemaphoreType.DMA(((NZ - 1) * NSUB,)),
            pltpu.SemaphoreType.DMA((NZ - 1,)),
            pltpu.SemaphoreType.DMA((NZ - 1,)),
            pltpu.SemaphoreType.DMA((8,)),
            pltpu.SemaphoreType.DMA((8,)),
            pltpu.SemaphoreType.DMA((8,)),
            pltpu.SemaphoreType.DMA((8,)),
            pltpu.SemaphoreType.DMA((2,)),
        ],
        compiler_params=pltpu.CompilerParams(collective_id=0),
    )(x)
